# baseline (speedup 1.0000x reference)
import jax
import jax.numpy as jnp
from jax.experimental import pallas as pl
from jax.experimental.pallas import tpu as pltpu

_BB = 32
_J = 20
_DEPTH = 1000
_ROW = _DEPTH * _J
_LANES = 128
_BLK_ROWS = _BB * _ROW // _LANES


def _onehot_body(x_ref, o_ref):
    o_ref[...] = jnp.zeros_like(o_ref)
    lane = jax.lax.broadcasted_iota(jnp.int32, (1, _LANES), 1)
    for bi in range(_BB):
        for j in range(_J):
            t = bi * _ROW + x_ref[bi, j] * _J + j
            row = (lane == t % _LANES).astype(jnp.float32)
            o_ref[pl.ds(t // _LANES, 1), :] += row


def kernel(X_in, ones):
    B, J = X_in.shape
    depth = ones.shape[0]
    flat = pl.pallas_call(
        _onehot_body,
        grid=(B // _BB,),
        in_specs=[pl.BlockSpec((_BB, J), lambda i: (i, 0),
                               memory_space=pltpu.SMEM)],
        out_specs=pl.BlockSpec((_BLK_ROWS, _LANES), lambda i: (i, 0)),
        out_shape=jax.ShapeDtypeStruct((B * depth * J // _LANES, _LANES),
                                       jnp.float32),
    )(X_in)
    return flat.reshape(B, depth, J)


# speedup vs baseline: 24.1442x; 24.1442x over previous
import jax
import jax.numpy as jnp
from jax.experimental import pallas as pl

_DBLK = 8


def _onehot_t_body(xt_ref, o_ref):
    j, dblk, b = o_ref.shape
    d0 = pl.program_id(0) * dblk
    d = jax.lax.broadcasted_iota(jnp.int32, (j, dblk, b), 1) + d0
    x = xt_ref[...]
    o_ref[...] = (x[:, None, :] == d).astype(jnp.float32)


def kernel(X_in, ones):
    B, J = X_in.shape
    depth = ones.shape[0]
    xt = X_in.T
    t = pl.pallas_call(
        _onehot_t_body,
        grid=(depth // _DBLK,),
        in_specs=[pl.BlockSpec((J, B), lambda i: (0, 0))],
        out_specs=pl.BlockSpec((J, _DBLK, B), lambda i: (0, i, 0)),
        out_shape=jax.ShapeDtypeStruct((J, depth, B), jnp.float32),
    )(xt)
    return jnp.transpose(t, (2, 1, 0))


# speedup vs baseline: 56.4343x; 2.3374x over previous
import jax
import jax.numpy as jnp
from jax.experimental import pallas as pl

_DBLK = 40


def _onehot_t_body(xt_ref, o_ref):
    j, dblk, b = o_ref.shape
    d0 = pl.program_id(0) * dblk
    d = jax.lax.broadcasted_iota(jnp.int32, (j, dblk, b), 1) + d0
    x = xt_ref[...]
    o_ref[...] = (x[:, None, :] == d).astype(jnp.float32)


def kernel(X_in, ones):
    B, J = X_in.shape
    depth = ones.shape[0]
    xt = X_in.T
    t = pl.pallas_call(
        _onehot_t_body,
        grid=(depth // _DBLK,),
        in_specs=[pl.BlockSpec((J, B), lambda i: (0, 0))],
        out_specs=pl.BlockSpec((J, _DBLK, B), lambda i: (0, i, 0)),
        out_shape=jax.ShapeDtypeStruct((J, depth, B), jnp.float32),
    )(xt)
    return jnp.transpose(t, (2, 1, 0))
